# Initial kernel scaffold; baseline (speedup 1.0000x reference)
import jax
import jax.numpy as jnp
from jax.experimental import pallas as pl


def kernel(lap_indices, lap_values, trust_indices, trust_values, add_indices, add_values, features, W_lin, b_lin, W_iat, b_iat, W_am, b_am, W_aa, b_aa, W_at, b_at, a_main, a_add, a_trust):
    raise NotImplementedError("write your pallas kernel here")


# speedup vs baseline: 5.2422x; 5.2422x over previous
import functools

import jax
import jax.numpy as jnp
from jax import lax
from jax.experimental import pallas as pl
from jax.experimental.pallas import tpu as pltpu
from jax.experimental.pallas import tpu_sc as plsc

NC = 2
NS = 16
LANES = 16


def _dotT(x, w):
    return lax.dot_general(x, w, (((1,), (1,)), ((), ())),
                           preferred_element_type=jnp.float32)


def _prep_body(f_ref, wl_ref, wi_ref, bsum_ref, z_ref, base_ref):
    f = f_ref[...]
    p1 = _dotT(f, wl_ref[...])
    p2 = _dotT(f * f, wi_ref[...])
    z_ref[...] = p1 + p2
    base_ref[...] = p1 + bsum_ref[...]


def _make_prep(n, d, r):
    return pl.pallas_call(
        _prep_body,
        grid=(n // r,),
        in_specs=[
            pl.BlockSpec((r, d), lambda i: (i, 0)),
            pl.BlockSpec((d, d), lambda i: (0, 0)),
            pl.BlockSpec((d, d), lambda i: (0, 0)),
            pl.BlockSpec((1, d), lambda i: (0, 0)),
        ],
        out_specs=[pl.BlockSpec((r, d), lambda i: (i, 0)),
                   pl.BlockSpec((r, d), lambda i: (i, 0))],
        out_shape=[jax.ShapeDtypeStruct((n, d), jnp.float32),
                   jax.ShapeDtypeStruct((n, d), jnp.float32)],
    )


def _make_spmm3(n, e, d):
    nw = NC * NS
    epw = e // nw
    K = 80
    nchunk = epw // K
    rpt = n // NS
    zrows = 125
    nzero = rpt // zrows
    assert epw * nw == e and nchunk * K == epw and nzero * zrows == rpt
    mesh = plsc.VectorSubcoreMesh(core_axis_name="c", subcore_axis_name="s")

    @functools.partial(
        pl.kernel,
        mesh=mesh,
        out_type=jax.ShapeDtypeStruct((6, n, d), jnp.float32),
        scratch_types=[
            pltpu.VMEM((K,), jnp.int32),
            pltpu.VMEM((K,), jnp.int32),
            pltpu.VMEM((K,), jnp.float32),
            pltpu.VMEM((K, d), jnp.float32),
            pltpu.VMEM((zrows, d), jnp.float32),
            pltpu.VMEM_SHARED((n, d), jnp.float32),
            pltpu.SemaphoreType.DMA,
        ],
    )
    def spmm3(r0, c0, v0, r1, c1, v1, r2, c2, v2, z_hbm, out_hbm,
              row_v, col_v, val_v, rows_v, zbuf, acc, sem):
        cid = lax.axis_index("c")
        sid = lax.axis_index("s")
        wid = cid * NS + sid

        def zb(i, carry):
            for j in range(d // LANES):
                zbuf[i, pl.ds(j * LANES, LANES)] = jnp.zeros((LANES,),
                                                             jnp.float32)
            return carry
        lax.fori_loop(0, zrows, zb, 0)

        for b, (rh, ch, vh) in enumerate(
                ((r0, c0, v0), (r1, c1, v1), (r2, c2, v2))):
            for i in range(nzero):
                pltpu.sync_copy(zbuf,
                                acc.at[pl.ds(sid * rpt + i * zrows, zrows)])
            plsc.subcore_barrier()

            def chunk(ci, carry):
                ebase = pl.multiple_of(wid * epw + ci * K, 8)
                pltpu.sync_copy(rh.at[pl.ds(ebase, K)], row_v)
                pltpu.sync_copy(ch.at[pl.ds(ebase, K)], col_v)
                pltpu.sync_copy(vh.at[pl.ds(ebase, K)], val_v)
                pltpu.async_copy(z_hbm.at[col_v], rows_v, sem).wait()

                def group(g, ecarry):
                    v16 = val_v[pl.ds(g * LANES, LANES)]
                    for l in range(LANES):
                        idx = jnp.full((LANES,), l, jnp.int32)
                        vspl = v16.at[idx].get(mode="promise_in_bounds",
                                               unique_indices=False)
                        for j in range(d // LANES):
                            sl = (g * LANES + l, pl.ds(j * LANES, LANES))
                            rows_v[sl] = rows_v[sl] * vspl
                    return ecarry
                lax.fori_loop(0, K // LANES, group, 0)

                pltpu.sync_copy(rows_v, acc.at[row_v], add=True)
                return carry
            lax.fori_loop(0, nchunk, chunk, 0)
            plsc.subcore_barrier()

            @pl.when(sid == 0)
            def _():
                pltpu.sync_copy(acc, out_hbm.at[2 * b + cid])
            plsc.subcore_barrier()

    return spmm3


def _attn_body(sp_ref, base_ref, wa_ref, ba_ref, a_ref, out_ref):
    i = pl.program_id(0)

    @pl.when(i == 0)
    def _():
        out_ref[...] = jnp.zeros_like(out_ref)

    parts = []
    for b in range(3):
        msum = sp_ref[2 * b] + sp_ref[2 * b + 1] + base_ref[...]
        t = jnp.tanh(_dotT(msum, wa_ref[b]) + ba_ref[b][None, :])
        parts.append(jnp.sum(t * a_ref[b][None, :], axis=0))
    out_ref[...] = out_ref[...] + jnp.stack(parts, axis=0)


def _make_attn(n, d, r):
    return pl.pallas_call(
        _attn_body,
        grid=(n // r,),
        in_specs=[
            pl.BlockSpec((6, r, d), lambda i: (0, i, 0)),
            pl.BlockSpec((r, d), lambda i: (i, 0)),
            pl.BlockSpec((3, d, d), lambda i: (0, 0, 0)),
            pl.BlockSpec((3, d), lambda i: (0, 0)),
            pl.BlockSpec((3, d), lambda i: (0, 0)),
        ],
        out_specs=pl.BlockSpec((3, d), lambda i: (0, 0)),
        out_shape=jax.ShapeDtypeStruct((3, d), jnp.float32),
        compiler_params=pltpu.CompilerParams(
            dimension_semantics=("arbitrary",)),
    )


def _comb_body(sp_ref, base_ref, beta_ref, out_ref):
    be = beta_ref[...]
    s0 = sp_ref[0] + sp_ref[1]
    s1 = sp_ref[2] + sp_ref[3]
    s2 = sp_ref[4] + sp_ref[5]
    out_ref[...] = (base_ref[...] + be[0, 0] * s0 + be[0, 1] * s1
                    + be[0, 2] * s2)


def _make_comb(n, d, r):
    return pl.pallas_call(
        _comb_body,
        grid=(n // r,),
        in_specs=[
            pl.BlockSpec((6, r, d), lambda i: (0, i, 0)),
            pl.BlockSpec((r, d), lambda i: (i, 0)),
            pl.BlockSpec((1, d), lambda i: (0, 0)),
        ],
        out_specs=pl.BlockSpec((r, d), lambda i: (i, 0)),
        out_shape=jax.ShapeDtypeStruct((n, d), jnp.float32),
    )


def kernel(lap_indices, lap_values, trust_indices, trust_values,
           add_indices, add_values, features, W_lin, b_lin, W_iat, b_iat,
           W_am, b_am, W_aa, b_aa, W_at, b_at, a_main, a_add, a_trust):
    n, d = features.shape
    e = lap_values.shape[0]
    r = 2000

    bsum = (b_lin + b_iat).reshape(1, d)
    z, base = _make_prep(n, d, r)(features, W_lin, W_iat, bsum)

    i32 = jnp.int32
    sp = _make_spmm3(n, e, d)(
        lap_indices[0].astype(i32), lap_indices[1].astype(i32), lap_values,
        add_indices[0].astype(i32), add_indices[1].astype(i32), add_values,
        trust_indices[0].astype(i32), trust_indices[1].astype(i32),
        trust_values, z)

    wa = jnp.stack([W_am, W_aa, W_at])
    ba = jnp.stack([b_am, b_aa, b_at])
    av = jnp.stack([a_main[:, 0], a_add[:, 0], a_trust[:, 0]])
    colsums = _make_attn(n, d, r)(sp, base, wa, ba, av)
    w = colsums.sum(axis=1) / n
    beta = jax.nn.softmax(w)
    beta128 = jnp.zeros((1, d), jnp.float32).at[0, :3].set(beta)

    return _make_comb(n, d, r)(sp, base, beta128)


# speedup vs baseline: 10.5601x; 2.0144x over previous
import functools

import jax
import jax.numpy as jnp
from jax import lax
from jax.experimental import pallas as pl
from jax.experimental.pallas import tpu as pltpu
from jax.experimental.pallas import tpu_sc as plsc

NC = 2
NS = 16
LANES = 16


def _dotT(x, w):
    return lax.dot_general(x, w, (((1,), (1,)), ((), ())),
                           preferred_element_type=jnp.float32)


def _prep_body(f_ref, wl_ref, wi_ref, bsum_ref, z_ref, base_ref):
    f = f_ref[...]
    p1 = _dotT(f, wl_ref[...])
    p2 = _dotT(f * f, wi_ref[...])
    z_ref[...] = p1 + p2
    base_ref[...] = p1 + bsum_ref[...]


def _make_prep(n, d, r):
    return pl.pallas_call(
        _prep_body,
        grid=(n // r,),
        in_specs=[
            pl.BlockSpec((r, d), lambda i: (i, 0)),
            pl.BlockSpec((d, d), lambda i: (0, 0)),
            pl.BlockSpec((d, d), lambda i: (0, 0)),
            pl.BlockSpec((1, d), lambda i: (0, 0)),
        ],
        out_specs=[pl.BlockSpec((r, d), lambda i: (i, 0)),
                   pl.BlockSpec((r, d), lambda i: (i, 0))],
        out_shape=[jax.ShapeDtypeStruct((n, d), jnp.float32),
                   jax.ShapeDtypeStruct((n, d), jnp.float32)],
    )


def _make_spmm3(n, e, d):
    nw = NC * NS
    epw = e // nw
    K = 80
    nchunk = epw // K
    rpt = n // NS
    nzf = rpt // K
    zrem = rpt - nzf * K
    assert epw * nw == e and nchunk * K == epw
    assert K % LANES == 0 and nchunk % 2 == 1 and nchunk >= 3
    npair = (nchunk - 1) // 2
    mesh = plsc.VectorSubcoreMesh(core_axis_name="c", subcore_axis_name="s")

    @functools.partial(
        pl.kernel,
        mesh=mesh,
        out_type=jax.ShapeDtypeStruct((6, n, d), jnp.float32),
        scratch_types=[
            pltpu.VMEM((K, d), jnp.float32),
            pltpu.VMEM((K, d), jnp.float32),
            pltpu.VMEM((2, K), jnp.int32),
            pltpu.VMEM((2, K), jnp.int32),
            pltpu.VMEM((K,), jnp.float32),
            pltpu.VMEM((K,), jnp.float32),
            pltpu.VMEM_SHARED((n, d), jnp.float32),
            pltpu.SemaphoreType.DMA,
            pltpu.SemaphoreType.DMA,
            pltpu.SemaphoreType.DMA,
            pltpu.SemaphoreType.DMA,
        ],
    )
    def spmm3(i0, v0, i1, v1, i2, v2, z_hbm, out_hbm,
              bufa, bufb, cba, cbb, cva, cvb, acc, sga, sgb, sia, sib):
        cid = lax.axis_index("c")
        sid = lax.axis_index("s")
        wid = cid * NS + sid

        def gstart(cb, buf, sem):
            pltpu.async_copy(z_hbm.at[cb.at[1]], buf, sem)

        def gwait(cb, buf, sem):
            pltpu.make_async_copy(z_hbm.at[cb.at[1]], buf, sem).wait()

        def process(buf, cb, cv):
            def group(g, ecarry):
                v16 = cv[pl.ds(g * LANES, LANES)]
                for l in range(LANES):
                    idx = jnp.full((LANES,), l, jnp.int32)
                    vspl = v16.at[idx].get(mode="promise_in_bounds",
                                           unique_indices=False)
                    for j in range(d // LANES):
                        sl = (g * LANES + l, pl.ds(j * LANES, LANES))
                        buf[sl] = buf[sl] * vspl
                return ecarry
            lax.fori_loop(0, K // LANES, group, 0)
            pltpu.sync_copy(buf, acc.at[cb.at[0]], add=True)

        for b, (iall, vall) in enumerate(((i0, v0), (i1, v1), (i2, v2))):
            ih = iall.at[wid]
            vh = vall.at[wid]

            def istart(c, cbi, cbv, sem):
                pltpu.async_copy(ih.at[c], cbi, sem)
                pltpu.async_copy(vh.at[c], cbv, sem)

            def iwait(c, cbi, cbv, sem):
                pltpu.make_async_copy(ih.at[c], cbi, sem).wait()
                pltpu.make_async_copy(vh.at[c], cbv, sem).wait()

            def zb(i, carry):
                for j in range(d // LANES):
                    bufa[i, pl.ds(j * LANES, LANES)] = jnp.zeros(
                        (LANES,), jnp.float32)
                return carry
            lax.fori_loop(0, K, zb, 0)
            zbase = sid * rpt
            for i in range(nzf):
                pltpu.sync_copy(bufa, acc.at[pl.ds(zbase + i * K, K)])
            if zrem:
                pltpu.sync_copy(bufa.at[pl.ds(0, zrem)],
                                acc.at[pl.ds(zbase + nzf * K, zrem)])
            plsc.subcore_barrier()

            istart(0, cbb, cvb, sib)
            iwait(0, cbb, cvb, sib)
            gstart(cbb, bufb, sgb)
            istart(1, cba, cva, sia)
            iwait(1, cba, cva, sia)
            gwait(cbb, bufb, sgb)
            gstart(cba, bufa, sga)
            process(bufb, cbb, cvb)
            istart(2, cbb, cvb, sib)

            def pair(i, carry):
                ca = 1 + 2 * i
                cn = ca + 1
                gwait(cba, bufa, sga)
                iwait(cn, cbb, cvb, sib)
                gstart(cbb, bufb, sgb)
                process(bufa, cba, cva)

                @pl.when(ca + 2 < nchunk)
                def _():
                    istart(ca + 2, cba, cva, sia)

                gwait(cbb, bufb, sgb)

                @pl.when(cn + 1 < nchunk)
                def _():
                    iwait(cn + 1, cba, cva, sia)
                    gstart(cba, bufa, sga)

                process(bufb, cbb, cvb)

                @pl.when(cn + 2 < nchunk)
                def _():
                    istart(cn + 2, cbb, cvb, sib)
                return carry
            lax.fori_loop(0, npair, pair, 0)
            plsc.subcore_barrier()

            @pl.when(sid == 0)
            def _():
                pltpu.sync_copy(acc, out_hbm.at[2 * b + cid])
            plsc.subcore_barrier()

    return spmm3


def _attn_body(sp_ref, base_ref, wa_ref, ba_ref, a_ref, out_ref):
    i = pl.program_id(0)

    @pl.when(i == 0)
    def _():
        out_ref[...] = jnp.zeros_like(out_ref)

    parts = []
    for b in range(3):
        msum = sp_ref[2 * b] + sp_ref[2 * b + 1] + base_ref[...]
        t = jnp.tanh(_dotT(msum, wa_ref[b]) + ba_ref[b][None, :])
        parts.append(jnp.sum(t * a_ref[b][None, :], axis=0))
    out_ref[...] = out_ref[...] + jnp.stack(parts, axis=0)


def _make_attn(n, d, r):
    return pl.pallas_call(
        _attn_body,
        grid=(n // r,),
        in_specs=[
            pl.BlockSpec((6, r, d), lambda i: (0, i, 0)),
            pl.BlockSpec((r, d), lambda i: (i, 0)),
            pl.BlockSpec((3, d, d), lambda i: (0, 0, 0)),
            pl.BlockSpec((3, d), lambda i: (0, 0)),
            pl.BlockSpec((3, d), lambda i: (0, 0)),
        ],
        out_specs=pl.BlockSpec((3, d), lambda i: (0, 0)),
        out_shape=jax.ShapeDtypeStruct((3, d), jnp.float32),
        compiler_params=pltpu.CompilerParams(
            dimension_semantics=("arbitrary",)),
    )


def _comb_body(sp_ref, base_ref, beta_ref, out_ref):
    be = beta_ref[...]
    s0 = sp_ref[0] + sp_ref[1]
    s1 = sp_ref[2] + sp_ref[3]
    s2 = sp_ref[4] + sp_ref[5]
    out_ref[...] = (base_ref[...] + be[0, 0] * s0 + be[0, 1] * s1
                    + be[0, 2] * s2)


def _make_comb(n, d, r):
    return pl.pallas_call(
        _comb_body,
        grid=(n // r,),
        in_specs=[
            pl.BlockSpec((6, r, d), lambda i: (0, i, 0)),
            pl.BlockSpec((r, d), lambda i: (i, 0)),
            pl.BlockSpec((1, d), lambda i: (0, 0)),
        ],
        out_specs=pl.BlockSpec((r, d), lambda i: (i, 0)),
        out_shape=jax.ShapeDtypeStruct((n, d), jnp.float32),
    )


def kernel(lap_indices, lap_values, trust_indices, trust_values,
           add_indices, add_values, features, W_lin, b_lin, W_iat, b_iat,
           W_am, b_am, W_aa, b_aa, W_at, b_at, a_main, a_add, a_trust):
    n, d = features.shape
    e = lap_values.shape[0]
    r = 2000

    bsum = (b_lin + b_iat).reshape(1, d)
    z, base = _make_prep(n, d, r)(features, W_lin, W_iat, bsum)

    i32 = jnp.int32
    nw = NC * NS
    kk = 80
    nch = e // (nw * kk)

    def _pidx(idx):
        arr = jnp.stack([idx[0].astype(i32), idx[1].astype(i32)], axis=0)
        return arr.reshape(2, nw, nch, kk).transpose(1, 2, 0, 3)

    def _pval(vals):
        return vals.reshape(nw, nch, kk)

    sp = _make_spmm3(n, e, d)(
        _pidx(lap_indices), _pval(lap_values),
        _pidx(add_indices), _pval(add_values),
        _pidx(trust_indices), _pval(trust_values), z)

    wa = jnp.stack([W_am, W_aa, W_at])
    ba = jnp.stack([b_am, b_aa, b_at])
    av = jnp.stack([a_main[:, 0], a_add[:, 0], a_trust[:, 0]])
    colsums = _make_attn(n, d, r)(sp, base, wa, ba, av)
    w = colsums.sum(axis=1) / n
    beta = jax.nn.softmax(w)
    beta128 = jnp.zeros((1, d), jnp.float32).at[0, :3].set(beta)

    return _make_comb(n, d, r)(sp, base, beta128)


# speedup vs baseline: 12.1777x; 1.1532x over previous
import functools

import jax
import jax.numpy as jnp
from jax import lax
from jax.experimental import pallas as pl
from jax.experimental.pallas import tpu as pltpu
from jax.experimental.pallas import tpu_sc as plsc

NC = 2
NS = 16
LANES = 16


def _dotT(x, w):
    return lax.dot_general(x, w, (((1,), (1,)), ((), ())),
                           preferred_element_type=jnp.float32)


def _prep_body(f_ref, wl_ref, wi_ref, bsum_ref, z_ref, base_ref):
    f = f_ref[...]
    p1 = _dotT(f, wl_ref[...])
    p2 = _dotT(f * f, wi_ref[...])
    z_ref[...] = p1 + p2
    base_ref[...] = p1 + bsum_ref[...]


def _make_prep(n, d, r):
    return pl.pallas_call(
        _prep_body,
        grid=(n // r,),
        in_specs=[
            pl.BlockSpec((r, d), lambda i: (i, 0)),
            pl.BlockSpec((d, d), lambda i: (0, 0)),
            pl.BlockSpec((d, d), lambda i: (0, 0)),
            pl.BlockSpec((1, d), lambda i: (0, 0)),
        ],
        out_specs=[pl.BlockSpec((r, d), lambda i: (i, 0)),
                   pl.BlockSpec((r, d), lambda i: (i, 0))],
        out_shape=[jax.ShapeDtypeStruct((n, d), jnp.float32),
                   jax.ShapeDtypeStruct((n, d), jnp.float32)],
    )


def _make_spmm3(n, e, d):
    nw = NC * NS
    epw = e // nw
    K = 80
    nchunk = epw // K
    rpt = n // NS
    nzf = rpt // K
    zrem = rpt - nzf * K
    assert epw * nw == e and nchunk * K == epw
    assert K % LANES == 0 and nchunk % 2 == 1 and nchunk >= 3
    npair = (nchunk - 1) // 2
    mesh = plsc.VectorSubcoreMesh(core_axis_name="c", subcore_axis_name="s")

    @functools.partial(
        pl.kernel,
        mesh=mesh,
        out_type=jax.ShapeDtypeStruct((6, n, d), jnp.float32),
        scratch_types=[
            pltpu.VMEM((K, d), jnp.float32),
            pltpu.VMEM((K, d), jnp.float32),
            pltpu.VMEM((2, K), jnp.int32),
            pltpu.VMEM((2, K), jnp.int32),
            pltpu.VMEM((nchunk, K), jnp.float32),
            pltpu.VMEM((K,), jnp.int32),
            pltpu.VMEM((K,), jnp.int32),
            pltpu.VMEM_SHARED((n, d), jnp.float32),
            pltpu.SemaphoreType.DMA,
            pltpu.SemaphoreType.DMA,
            pltpu.SemaphoreType.DMA,
            pltpu.SemaphoreType.DMA,
            pltpu.SemaphoreType.DMA,
            pltpu.SemaphoreType.DMA,
        ],
    )
    def spmm3(i0, v0, i1, v1, i2, v2, z_hbm, out_hbm,
              bufa, bufb, cba, cbb, vals, sidxa, sidxb, acc,
              sga, sgb, sia, sib, ssa, ssb):
        cid = lax.axis_index("c")
        sid = lax.axis_index("s")
        wid = cid * NS + sid

        def gstart(cb, buf, sem):
            pltpu.async_copy(z_hbm.at[cb.at[1]], buf, sem)

        def gwait(cb, buf, sem):
            pltpu.make_async_copy(z_hbm.at[cb.at[1]], buf, sem).wait()

        def sstart(buf, sidx, sem):
            pltpu.async_copy(buf, acc.at[sidx], sem, add=True)

        def swait(buf, sidx, sem):
            pltpu.make_async_copy(buf, acc.at[sidx], sem).wait()

        def rowcopy(cb, sidx):
            for g in range(K // LANES):
                sidx[pl.ds(g * LANES, LANES)] = cb[0, pl.ds(g * LANES, LANES)]

        def mul(buf, c):
            def group(g, ecarry):
                v16 = vals[c, pl.ds(g * LANES, LANES)]
                for l in range(LANES):
                    idx = jnp.full((LANES,), l, jnp.int32)
                    vspl = v16.at[idx].get(mode="promise_in_bounds",
                                           unique_indices=False)
                    for j in range(d // LANES):
                        sl = (g * LANES + l, pl.ds(j * LANES, LANES))
                        buf[sl] = buf[sl] * vspl
                return ecarry
            lax.fori_loop(0, K // LANES, group, 0)

        for b, (iall, vall) in enumerate(((i0, v0), (i1, v1), (i2, v2))):
            ih = iall.at[wid]

            def istart(c, cbi, sem):
                pltpu.async_copy(ih.at[c], cbi, sem)

            def iwait(c, cbi, sem):
                pltpu.make_async_copy(ih.at[c], cbi, sem).wait()

            def zb(i, carry):
                for j in range(d // LANES):
                    bufa[i, pl.ds(j * LANES, LANES)] = jnp.zeros(
                        (LANES,), jnp.float32)
                return carry
            lax.fori_loop(0, K, zb, 0)
            zbase = sid * rpt
            pltpu.async_copy(vall.at[wid], vals, sib)
            for i in range(nzf):
                pltpu.async_copy(bufa, acc.at[pl.ds(zbase + i * K, K)], sia)
            if zrem:
                pltpu.async_copy(bufa.at[pl.ds(0, zrem)],
                                 acc.at[pl.ds(zbase + nzf * K, zrem)], sia)
            for i in range(nzf):
                pltpu.make_async_copy(
                    bufa, acc.at[pl.ds(zbase + i * K, K)], sia).wait()
            if zrem:
                pltpu.make_async_copy(
                    bufa.at[pl.ds(0, zrem)],
                    acc.at[pl.ds(zbase + nzf * K, zrem)], sia).wait()
            pltpu.make_async_copy(vall.at[wid], vals, sib).wait()
            plsc.subcore_barrier()

            istart(0, cbb, sib)
            iwait(0, cbb, sib)
            gstart(cbb, bufb, sgb)
            istart(1, cba, sia)
            gwait(cbb, bufb, sgb)
            rowcopy(cbb, sidxb)
            istart(2, cbb, sib)
            mul(bufb, 0)
            sstart(bufb, sidxb, ssb)
            iwait(1, cba, sia)
            gstart(cba, bufa, sga)

            def pair(i, carry):
                ca = 1 + 2 * i
                cn = ca + 1
                gwait(cba, bufa, sga)
                iwait(cn, cbb, sib)
                swait(bufb, sidxb, ssb)
                gstart(cbb, bufb, sgb)
                rowcopy(cba, sidxa)

                @pl.when(ca + 2 < nchunk)
                def _():
                    istart(ca + 2, cba, sia)

                mul(bufa, ca)
                sstart(bufa, sidxa, ssa)

                gwait(cbb, bufb, sgb)
                swait(bufa, sidxa, ssa)

                @pl.when(cn + 1 < nchunk)
                def _():
                    iwait(cn + 1, cba, sia)
                    gstart(cba, bufa, sga)

                rowcopy(cbb, sidxb)

                @pl.when(cn + 2 < nchunk)
                def _():
                    istart(cn + 2, cbb, sib)

                mul(bufb, cn)
                sstart(bufb, sidxb, ssb)
                return carry
            lax.fori_loop(0, npair, pair, 0)
            swait(bufb, sidxb, ssb)
            plsc.subcore_barrier()

            @pl.when(sid == 0)
            def _():
                pltpu.sync_copy(acc, out_hbm.at[2 * b + cid])
            plsc.subcore_barrier()

    return spmm3


def _attn_body(sp_ref, base_ref, wa_ref, ba_ref, a_ref, out_ref):
    i = pl.program_id(0)

    @pl.when(i == 0)
    def _():
        out_ref[...] = jnp.zeros_like(out_ref)

    parts = []
    for b in range(3):
        msum = sp_ref[2 * b] + sp_ref[2 * b + 1] + base_ref[...]
        t = jnp.tanh(_dotT(msum, wa_ref[b]) + ba_ref[b][None, :])
        parts.append(jnp.sum(t * a_ref[b][None, :], axis=0))
    out_ref[...] = out_ref[...] + jnp.stack(parts, axis=0)


def _make_attn(n, d, r):
    return pl.pallas_call(
        _attn_body,
        grid=(n // r,),
        in_specs=[
            pl.BlockSpec((6, r, d), lambda i: (0, i, 0)),
            pl.BlockSpec((r, d), lambda i: (i, 0)),
            pl.BlockSpec((3, d, d), lambda i: (0, 0, 0)),
            pl.BlockSpec((3, d), lambda i: (0, 0)),
            pl.BlockSpec((3, d), lambda i: (0, 0)),
        ],
        out_specs=pl.BlockSpec((3, d), lambda i: (0, 0)),
        out_shape=jax.ShapeDtypeStruct((3, d), jnp.float32),
        compiler_params=pltpu.CompilerParams(
            dimension_semantics=("arbitrary",)),
    )


def _comb_body(sp_ref, base_ref, beta_ref, out_ref):
    be = beta_ref[...]
    s0 = sp_ref[0] + sp_ref[1]
    s1 = sp_ref[2] + sp_ref[3]
    s2 = sp_ref[4] + sp_ref[5]
    out_ref[...] = (base_ref[...] + be[0, 0] * s0 + be[0, 1] * s1
                    + be[0, 2] * s2)


def _make_comb(n, d, r):
    return pl.pallas_call(
        _comb_body,
        grid=(n // r,),
        in_specs=[
            pl.BlockSpec((6, r, d), lambda i: (0, i, 0)),
            pl.BlockSpec((r, d), lambda i: (i, 0)),
            pl.BlockSpec((1, d), lambda i: (0, 0)),
        ],
        out_specs=pl.BlockSpec((r, d), lambda i: (i, 0)),
        out_shape=jax.ShapeDtypeStruct((n, d), jnp.float32),
    )


def kernel(lap_indices, lap_values, trust_indices, trust_values,
           add_indices, add_values, features, W_lin, b_lin, W_iat, b_iat,
           W_am, b_am, W_aa, b_aa, W_at, b_at, a_main, a_add, a_trust):
    n, d = features.shape
    e = lap_values.shape[0]
    r = 2000

    bsum = (b_lin + b_iat).reshape(1, d)
    z, base = _make_prep(n, d, r)(features, W_lin, W_iat, bsum)

    i32 = jnp.int32
    nw = NC * NS
    kk = 80
    nch = e // (nw * kk)

    def _pidx(idx):
        arr = jnp.stack([idx[0].astype(i32), idx[1].astype(i32)], axis=0)
        return arr.reshape(2, nw, nch, kk).transpose(1, 2, 0, 3)

    def _pval(vals):
        return vals.reshape(nw, nch, kk)

    sp = _make_spmm3(n, e, d)(
        _pidx(lap_indices), _pval(lap_values),
        _pidx(add_indices), _pval(add_values),
        _pidx(trust_indices), _pval(trust_values), z)

    wa = jnp.stack([W_am, W_aa, W_at])
    ba = jnp.stack([b_am, b_aa, b_at])
    av = jnp.stack([a_main[:, 0], a_add[:, 0], a_trust[:, 0]])
    colsums = _make_attn(n, d, r)(sp, base, wa, ba, av)
    w = colsums.sum(axis=1) / n
    beta = jax.nn.softmax(w)
    beta128 = jnp.zeros((1, d), jnp.float32).at[0, :3].set(beta)

    return _make_comb(n, d, r)(sp, base, beta128)
